# Initial kernel scaffold; baseline (speedup 1.0000x reference)
import jax
import jax.numpy as jnp
from jax.experimental import pallas as pl


def kernel(x, edge_index, W1, a_src1, a_dst1, b1, Wm, a_srcm, a_dstm, bm, Wl, a_srcl, a_dstl, bl):
    raise NotImplementedError("write your pallas kernel here")


# speedup vs baseline: 8.4045x; 8.4045x over previous
import functools

import jax
import jax.numpy as jnp
from jax import lax
from jax.experimental import pallas as pl
from jax.experimental.pallas import tpu as pltpu
from jax.experimental.pallas import tpu_sc as plsc

N = 10000
NP = 10240
NROWS = NP // 128
E = 160000
EP = 163840
CPT = (EP // 128) // 16
STRIPE = NP // 16
DEN_STRIPE = 8

NBLK = 1024
NGRID = NP // NBLK


def _tc1_body(x_ref, w_ref, as_ref, ad_ref, h0, h1, h2, h3, os_ref, od_ref):
    h = jnp.dot(x_ref[...], w_ref[...], preferred_element_type=jnp.float32)
    h0[...] = h[:, 0:128]
    h1[...] = h[:, 128:256]
    h2[...] = h[:, 256:384]
    h3[...] = h[:, 384:512]
    os_ref[...] = jnp.dot(h, as_ref[...], preferred_element_type=jnp.float32)
    od_ref[...] = jnp.dot(h, ad_ref[...], preferred_element_type=jnp.float32)


def _dense1(x, W1, a_src1, a_dst1):
    return pl.pallas_call(
        _tc1_body,
        grid=(NGRID,),
        in_specs=[
            pl.BlockSpec((NBLK, 256), lambda i: (i, 0)),
            pl.BlockSpec((256, 512), lambda i: (0, 0)),
            pl.BlockSpec((512,), lambda i: (0,)),
            pl.BlockSpec((512,), lambda i: (0,)),
        ],
        out_specs=[pl.BlockSpec((NBLK, 128), lambda i: (i, 0))] * 4
        + [pl.BlockSpec((NBLK,), lambda i: (i,))] * 2,
        out_shape=[jax.ShapeDtypeStruct((NP, 128), jnp.float32)] * 4
        + [jax.ShapeDtypeStruct((NP,), jnp.float32)] * 2,
    )(x, W1, a_src1, a_dst1)


def _tc2_body(a0, a1, a2, a3, den_ref, b_ref, wm_ref, wl_ref,
              ams_ref, amd_ref, als_ref, ald_ref,
              hm0, hm1, hl0, hl1, oms, omd, ols, old):
    d = den_ref[...]
    r = jnp.where(d > 0.0, 1.0 / jnp.where(d > 0.0, d, 1.0), 0.0)
    acc = jnp.concatenate([a0[...], a1[...], a2[...], a3[...]], axis=1)
    h = jnp.maximum(acc * r[:, None] + b_ref[...][None, :], 0.0)
    hm = jnp.dot(h, wm_ref[...], preferred_element_type=jnp.float32)
    hl = jnp.dot(h, wl_ref[...], preferred_element_type=jnp.float32)
    hm0[...] = hm[:, 0:128]
    hm1[...] = hm[:, 128:256]
    hl0[...] = hl[:, 0:128]
    hl1[...] = hl[:, 128:256]
    oms[...] = jnp.dot(hm, ams_ref[...], preferred_element_type=jnp.float32)
    omd[...] = jnp.dot(hm, amd_ref[...], preferred_element_type=jnp.float32)
    ols[...] = jnp.dot(hl, als_ref[...], preferred_element_type=jnp.float32)
    old[...] = jnp.dot(hl, ald_ref[...], preferred_element_type=jnp.float32)


def _dense2(acc_slices, den, b1, Wm, Wl, ams, amd, als, ald):
    vec = pl.BlockSpec((256,), lambda i: (0,))
    return pl.pallas_call(
        _tc2_body,
        grid=(NGRID,),
        in_specs=[pl.BlockSpec((NBLK, 128), lambda i: (i, 0))] * 4
        + [
            pl.BlockSpec((NBLK,), lambda i: (i,)),
            pl.BlockSpec((512,), lambda i: (0,)),
            pl.BlockSpec((512, 256), lambda i: (0, 0)),
            pl.BlockSpec((512, 256), lambda i: (0, 0)),
            vec, vec, vec, vec,
        ],
        out_specs=[pl.BlockSpec((NBLK, 128), lambda i: (i, 0))] * 4
        + [pl.BlockSpec((NBLK,), lambda i: (i,))] * 4,
        out_shape=[jax.ShapeDtypeStruct((NP, 128), jnp.float32)] * 4
        + [jax.ShapeDtypeStruct((NP,), jnp.float32)] * 4,
    )(*acc_slices, den, b1, Wm, Wl, ams, amd, als, ald)


def _tc3_body(m0, m1, l0, l1, dm_ref, dl_ref, bm_ref, bl_ref, mu_ref, ls_ref):
    dm = dm_ref[...]
    rm = jnp.where(dm > 0.0, 1.0 / jnp.where(dm > 0.0, dm, 1.0), 0.0)
    dl = dl_ref[...]
    rl = jnp.where(dl > 0.0, 1.0 / jnp.where(dl > 0.0, dl, 1.0), 0.0)
    accm = jnp.concatenate([m0[...], m1[...]], axis=1)
    accl = jnp.concatenate([l0[...], l1[...]], axis=1)
    mu_ref[...] = accm * rm[:, None] + bm_ref[...][None, :]
    ls_ref[...] = accl * rl[:, None] + bl_ref[...][None, :]


def _finalize(am, al, den_m, den_l, bm, bl):
    return pl.pallas_call(
        _tc3_body,
        grid=(NGRID,),
        in_specs=[pl.BlockSpec((NBLK, 128), lambda i: (i, 0))] * 4
        + [pl.BlockSpec((NBLK,), lambda i: (i,))] * 2
        + [pl.BlockSpec((256,), lambda i: (0,))] * 2,
        out_specs=[pl.BlockSpec((NBLK, 256), lambda i: (i, 0))] * 2,
        out_shape=[jax.ShapeDtypeStruct((N, 256), jnp.float32)] * 2,
    )(am[0], am[1], al[0], al[1], den_m, den_l, bm, bl)


SEG = 5
SEGC = CPT // SEG


def _phase_weights(s, tabs_hbm, tabd_hbm, w_hbm, den_hbm, srcr, dstr,
                   src_vm, dst_vm, w_vm, tab_s, tab_d, zeros1d, den_sp,
                   write_den):
    pltpu.sync_copy(tabs_hbm, tab_s)
    pltpu.sync_copy(tabd_hbm, tab_d)

    @pl.when(s < 10)
    def _zero_den():
        pltpu.sync_copy(zeros1d.at[pl.ds(1024 * s, 1024)],
                        den_sp.at[pl.ds(1024 * s, 1024)])
    plsc.subcore_barrier()

    for seg in range(SEG):
        base_row = s * CPT + seg * SEGC
        pltpu.sync_copy(srcr.at[pl.ds(base_row, SEGC)], src_vm)
        pltpu.sync_copy(dstr.at[pl.ds(base_row, SEGC)], dst_vm)

        def chunk(j, _):
            base_e = (base_row + j) * 128

            def vec(k, _):
                si = src_vm[j, pl.ds(k * 16, 16)]
                di = dst_vm[j, pl.ds(k * 16, 16)]
                av = plsc.load_gather(
                    tab_s, [jnp.right_shift(si, 7), jnp.bitwise_and(si, 127)])
                bv = plsc.load_gather(
                    tab_d, [jnp.right_shift(di, 7), jnp.bitwise_and(di, 127)])
                e = av + bv
                e = jnp.where(e > 0.0, e, 0.2 * e)
                w = jnp.exp(e)
                eid = base_e + k * 16 + lax.iota(jnp.int32, 16)
                w = jnp.where(eid < E, w, 0.0)
                w_vm[j, pl.ds(k * 16, 16)] = w
                return 0

            lax.fori_loop(0, 8, vec, 0)
            pltpu.sync_copy(w_vm.at[j], den_sp.at[dst_vm.at[j]], add=True)
            return 0

        lax.fori_loop(0, SEGC, chunk, 0)
        pltpu.sync_copy(w_vm, w_hbm.at[pl.ds(base_row, SEGC)])

    plsc.subcore_barrier()
    if write_den:
        @pl.when(s < 10)
        def _write_den():
            pltpu.sync_copy(den_sp.at[pl.ds(1024 * s, 1024)],
                            den_hbm.at[pl.ds(1024 * s, 1024)])


def _phase_slice(s, h_hbm, acc_hbm, w_hbm, srcr, dstr,
                 src_vm, dst_vm, w_vm, rows_vm, acc_sp, zeros2d):
    for z in range(STRIPE // 128):
        pltpu.sync_copy(zeros2d, acc_sp.at[pl.ds(STRIPE * s + 128 * z, 128)])
    plsc.subcore_barrier()

    for seg in range(SEG):
        base_row = s * CPT + seg * SEGC
        pltpu.sync_copy(srcr.at[pl.ds(base_row, SEGC)], src_vm)
        pltpu.sync_copy(dstr.at[pl.ds(base_row, SEGC)], dst_vm)
        pltpu.sync_copy(w_hbm.at[pl.ds(base_row, SEGC)], w_vm)

        def chunk(j, _):
            pltpu.sync_copy(h_hbm.at[src_vm.at[j]], rows_vm)

            def sgroup(g, _):
                wv = w_vm[j, pl.ds(g * 16, 16)]
                for i in range(16):
                    row = g * 16 + i
                    wvi = jnp.full((16,), wv[i], jnp.float32)
                    for k in range(8):
                        sl = pl.ds(k * 16, 16)
                        rows_vm[row, sl] = rows_vm[row, sl] * wvi
                return 0

            lax.fori_loop(0, 8, sgroup, 0)
            pltpu.sync_copy(rows_vm, acc_sp.at[dst_vm.at[j]], add=True)
            return 0

        lax.fori_loop(0, SEGC, chunk, 0)

    plsc.subcore_barrier()
    pltpu.sync_copy(
        acc_sp.at[pl.ds(STRIPE * s, STRIPE)],
        acc_hbm.at[pl.ds(STRIPE * s, STRIPE)],
    )


def _sc_body(h00, h01, h10, h11, tabs0, tabd0, tabs1, tabd1, srcr, dstr,
             zeros1d, zeros2d,
             acc00, acc01, acc10, acc11, den0, den1, w0_hbm, w1_hbm,
             src_vm, dst_vm, w_vm, tab_s, tab_d, rows_vm, den_sp, acc_sp,
             write_den1):
    c = lax.axis_index("c")
    s = lax.axis_index("s")

    @pl.when(c == 0)
    def _core0():
        _phase_weights(s, tabs0, tabd0, w0_hbm, den0, srcr, dstr,
                       src_vm, dst_vm, w_vm, tab_s, tab_d, zeros1d, den_sp,
                       True)
        _phase_slice(s, h00, acc00, w0_hbm, srcr, dstr,
                     src_vm, dst_vm, w_vm, rows_vm, acc_sp, zeros2d)
        _phase_slice(s, h01, acc01, w0_hbm, srcr, dstr,
                     src_vm, dst_vm, w_vm, rows_vm, acc_sp, zeros2d)

    @pl.when(c == 1)
    def _core1():
        _phase_weights(s, tabs1, tabd1, w1_hbm, den1, srcr, dstr,
                       src_vm, dst_vm, w_vm, tab_s, tab_d, zeros1d, den_sp,
                       write_den1)
        _phase_slice(s, h10, acc10, w1_hbm, srcr, dstr,
                     src_vm, dst_vm, w_vm, rows_vm, acc_sp, zeros2d)
        _phase_slice(s, h11, acc11, w1_hbm, srcr, dstr,
                     src_vm, dst_vm, w_vm, rows_vm, acc_sp, zeros2d)


def _sc_edge_pass(h_slices, tabs0, tabd0, tabs1, tabd1, srcr, dstr, write_den1):
    mesh = plsc.VectorSubcoreMesh(core_axis_name="c", subcore_axis_name="s")
    f32 = jnp.float32
    zeros1d = jnp.zeros((NP,), f32)
    zeros2d = jnp.zeros((128, 128), f32)
    kern = pl.kernel(
        functools.partial(_sc_body, write_den1=write_den1),
        out_type=[jax.ShapeDtypeStruct((NP, 128), f32)] * 4
        + [jax.ShapeDtypeStruct((NP,), f32)] * 2
        + [jax.ShapeDtypeStruct((EP // 128, 128), f32)] * 2,
        mesh=mesh,
        compiler_params=pltpu.CompilerParams(needs_layout_passes=False),
        scratch_types=[
            pltpu.VMEM((SEGC, 128), jnp.int32),
            pltpu.VMEM((SEGC, 128), jnp.int32),
            pltpu.VMEM((SEGC, 128), f32),
            pltpu.VMEM((NROWS, 128), f32),
            pltpu.VMEM((NROWS, 128), f32),
            pltpu.VMEM((128, 128), f32),
            pltpu.VMEM_SHARED((NP,), f32),
            pltpu.VMEM_SHARED((NP, 128), f32),
        ],
    )
    out = kern(h_slices[0], h_slices[1], h_slices[2], h_slices[3],
               tabs0, tabd0, tabs1, tabd1, srcr, dstr, zeros1d, zeros2d)
    return out[:6]


def kernel(x, edge_index, W1, a_src1, a_dst1, b1,
           Wm, a_srcm, a_dstm, bm, Wl, a_srcl, a_dstl, bl):
    src = edge_index[0]
    dst = edge_index[1]
    pad = jnp.zeros((EP - E,), jnp.int32)
    srcr = jnp.concatenate([src, pad]).reshape(EP // 128, 128)
    dstr = jnp.concatenate([dst, pad]).reshape(EP // 128, 128)

    h0, h1, h2, h3, a_s, a_d = _dense1(x, W1, a_src1, a_dst1)

    a_s = a_s.reshape(NROWS, 128)
    a_d = a_d.reshape(NROWS, 128)
    acc0, acc1, acc2, acc3, den1, _ = _sc_edge_pass(
        (h0, h1, h2, h3), a_s, a_d, a_s, a_d, srcr, dstr, False)

    (hm0, hm1, hl0, hl1, ams, amd, als, ald) = _dense2(
        (acc0, acc1, acc2, acc3), den1, b1, Wm, Wl,
        a_srcm, a_dstm, a_srcl, a_dstl)

    ams = ams.reshape(NROWS, 128)
    amd = amd.reshape(NROWS, 128)
    als = als.reshape(NROWS, 128)
    ald = ald.reshape(NROWS, 128)
    am0, am1, al0, al1, den_m, den_l = _sc_edge_pass(
        (hm0, hm1, hl0, hl1), ams, amd, als, ald, srcr, dstr, True)

    mu, logstd = _finalize((am0, am1), (al0, al1), den_m, den_l, bm, bl)
    return (mu, logstd)


# speedup vs baseline: 10.2731x; 1.2223x over previous
import functools

import jax
import jax.numpy as jnp
from jax import lax
from jax.experimental import pallas as pl
from jax.experimental.pallas import tpu as pltpu
from jax.experimental.pallas import tpu_sc as plsc

N = 10000
NP = 10240
NROWS = NP // 128
E = 160000
EP = 163840
CPT = (EP // 128) // 16
STRIPE = NP // 16
DEN_STRIPE = 8

NBLK = 1024
NGRID = NP // NBLK


def _tc1_body(x_ref, w_ref, as_ref, ad_ref, h0, h1, h2, h3, os_ref, od_ref):
    h = jnp.dot(x_ref[...], w_ref[...], preferred_element_type=jnp.float32)
    h0[...] = h[:, 0:128]
    h1[...] = h[:, 128:256]
    h2[...] = h[:, 256:384]
    h3[...] = h[:, 384:512]
    os_ref[...] = jnp.dot(h, as_ref[...], preferred_element_type=jnp.float32)
    od_ref[...] = jnp.dot(h, ad_ref[...], preferred_element_type=jnp.float32)


def _dense1(x, W1, a_src1, a_dst1):
    return pl.pallas_call(
        _tc1_body,
        grid=(NGRID,),
        in_specs=[
            pl.BlockSpec((NBLK, 256), lambda i: (i, 0)),
            pl.BlockSpec((256, 512), lambda i: (0, 0)),
            pl.BlockSpec((512,), lambda i: (0,)),
            pl.BlockSpec((512,), lambda i: (0,)),
        ],
        out_specs=[pl.BlockSpec((NBLK, 128), lambda i: (i, 0))] * 4
        + [pl.BlockSpec((NBLK,), lambda i: (i,))] * 2,
        out_shape=[jax.ShapeDtypeStruct((NP, 128), jnp.float32)] * 4
        + [jax.ShapeDtypeStruct((NP,), jnp.float32)] * 2,
    )(x, W1, a_src1, a_dst1)


def _tc2_body(a0, a1, a2, a3, den_ref, b_ref, wm_ref, wl_ref,
              ams_ref, amd_ref, als_ref, ald_ref,
              hm0, hm1, hl0, hl1, oms, omd, ols, old):
    d = den_ref[...]
    r = jnp.where(d > 0.0, 1.0 / jnp.where(d > 0.0, d, 1.0), 0.0)
    acc = jnp.concatenate([a0[...], a1[...], a2[...], a3[...]], axis=1)
    h = jnp.maximum(acc * r[:, None] + b_ref[...][None, :], 0.0)
    hm = jnp.dot(h, wm_ref[...], preferred_element_type=jnp.float32)
    hl = jnp.dot(h, wl_ref[...], preferred_element_type=jnp.float32)
    hm0[...] = hm[:, 0:128]
    hm1[...] = hm[:, 128:256]
    hl0[...] = hl[:, 0:128]
    hl1[...] = hl[:, 128:256]
    oms[...] = jnp.dot(hm, ams_ref[...], preferred_element_type=jnp.float32)
    omd[...] = jnp.dot(hm, amd_ref[...], preferred_element_type=jnp.float32)
    ols[...] = jnp.dot(hl, als_ref[...], preferred_element_type=jnp.float32)
    old[...] = jnp.dot(hl, ald_ref[...], preferred_element_type=jnp.float32)


def _dense2(acc_slices, den, b1, Wm, Wl, ams, amd, als, ald):
    vec = pl.BlockSpec((256,), lambda i: (0,))
    return pl.pallas_call(
        _tc2_body,
        grid=(NGRID,),
        in_specs=[pl.BlockSpec((NBLK, 128), lambda i: (i, 0))] * 4
        + [
            pl.BlockSpec((NBLK,), lambda i: (i,)),
            pl.BlockSpec((512,), lambda i: (0,)),
            pl.BlockSpec((512, 256), lambda i: (0, 0)),
            pl.BlockSpec((512, 256), lambda i: (0, 0)),
            vec, vec, vec, vec,
        ],
        out_specs=[pl.BlockSpec((NBLK, 128), lambda i: (i, 0))] * 4
        + [pl.BlockSpec((NBLK,), lambda i: (i,))] * 4,
        out_shape=[jax.ShapeDtypeStruct((NP, 128), jnp.float32)] * 4
        + [jax.ShapeDtypeStruct((NP,), jnp.float32)] * 4,
    )(*acc_slices, den, b1, Wm, Wl, ams, amd, als, ald)


def _tc3_body(m0, m1, l0, l1, dm_ref, dl_ref, bm_ref, bl_ref, mu_ref, ls_ref):
    dm = dm_ref[...]
    rm = jnp.where(dm > 0.0, 1.0 / jnp.where(dm > 0.0, dm, 1.0), 0.0)
    dl = dl_ref[...]
    rl = jnp.where(dl > 0.0, 1.0 / jnp.where(dl > 0.0, dl, 1.0), 0.0)
    accm = jnp.concatenate([m0[...], m1[...]], axis=1)
    accl = jnp.concatenate([l0[...], l1[...]], axis=1)
    mu_ref[...] = accm * rm[:, None] + bm_ref[...][None, :]
    ls_ref[...] = accl * rl[:, None] + bl_ref[...][None, :]


def _finalize(am, al, den_m, den_l, bm, bl):
    return pl.pallas_call(
        _tc3_body,
        grid=(NGRID,),
        in_specs=[pl.BlockSpec((NBLK, 128), lambda i: (i, 0))] * 4
        + [pl.BlockSpec((NBLK,), lambda i: (i,))] * 2
        + [pl.BlockSpec((256,), lambda i: (0,))] * 2,
        out_specs=[pl.BlockSpec((NBLK, 256), lambda i: (i, 0))] * 2,
        out_shape=[jax.ShapeDtypeStruct((N, 256), jnp.float32)] * 2,
    )(am[0], am[1], al[0], al[1], den_m, den_l, bm, bl)


SEG = 5
SEGC = CPT // SEG

TABD_OFF = 80


def _phase_weights(s, tabs_hbm, tabd_hbm, w_hbm, den_hbm, srcr, dstr,
                   src_vm, dst_vm, w_vm, u_vm, zeros1d, den_sp, densem,
                   write_den):
    pltpu.sync_copy(tabs_hbm, u_vm.at[pl.ds(0, NROWS)])
    pltpu.sync_copy(tabd_hbm, u_vm.at[pl.ds(TABD_OFF, NROWS)])

    @pl.when(s < 10)
    def _zero_den():
        pltpu.sync_copy(zeros1d.at[pl.ds(1024 * s, 1024)],
                        den_sp.at[pl.ds(1024 * s, 1024)])
    plsc.subcore_barrier()

    for seg in range(SEG):
        base_row = s * CPT + seg * SEGC
        pltpu.sync_copy(srcr.at[pl.ds(base_row, SEGC)], src_vm)
        pltpu.sync_copy(dstr.at[pl.ds(base_row, SEGC)], dst_vm)

        def chunk(j, _):
            base_e = (base_row + j) * 128

            def vec(k, _):
                si = src_vm[j, pl.ds(k * 16, 16)]
                di = dst_vm[j, pl.ds(k * 16, 16)]
                av = plsc.load_gather(
                    u_vm, [jnp.right_shift(si, 7), jnp.bitwise_and(si, 127)])
                bv = plsc.load_gather(
                    u_vm, [jnp.right_shift(di, 7) + TABD_OFF,
                           jnp.bitwise_and(di, 127)])
                e = av + bv
                e = jnp.where(e > 0.0, e, 0.2 * e)
                w = jnp.exp(e)
                eid = base_e + k * 16 + lax.iota(jnp.int32, 16)
                w = jnp.where(eid < E, w, 0.0)
                w_vm[j, pl.ds(k * 16, 16)] = w
                return 0

            lax.fori_loop(0, 8, vec, 0)
            pltpu.async_copy(w_vm.at[j], den_sp.at[dst_vm.at[j]], densem,
                             add=True)
            return 0

        lax.fori_loop(0, SEGC, chunk, 0)

        def drain(j, _):
            pltpu.make_async_copy(w_vm.at[j], den_sp.at[dst_vm.at[j]],
                                  densem).wait()
            return 0

        lax.fori_loop(0, SEGC, drain, 0)
        pltpu.sync_copy(w_vm, w_hbm.at[pl.ds(base_row, SEGC)])

    plsc.subcore_barrier()
    if write_den:
        @pl.when(s < 10)
        def _write_den():
            pltpu.sync_copy(den_sp.at[pl.ds(1024 * s, 1024)],
                            den_hbm.at[pl.ds(1024 * s, 1024)])


def _scale_rows(u_vm, boff, w_vm, j):
    def sgroup(g, _):
        wv = w_vm[j, pl.ds(g * 16, 16)]
        for i in range(16):
            row = g * 16 + i
            wvi = jnp.full((16,), wv[i], jnp.float32)
            for k in range(8):
                sl = pl.ds(k * 16, 16)
                u_vm[boff + row, sl] = u_vm[boff + row, sl] * wvi
        return 0

    lax.fori_loop(0, 8, sgroup, 0)


def _phase_slice(s, h_hbm, acc_hbm, w_hbm, srcr, dstr,
                 src_vm, dst_vm, w_vm, u_vm, acc_sp, zeros2d, gsemA, gsemB):
    bufA = u_vm.at[pl.ds(0, 128)]
    bufB = u_vm.at[pl.ds(128, 128)]
    for z in range(STRIPE // 128):
        pltpu.sync_copy(zeros2d, acc_sp.at[pl.ds(STRIPE * s + 128 * z, 128)])
    plsc.subcore_barrier()

    for seg in range(SEG):
        base_row = s * CPT + seg * SEGC
        pltpu.sync_copy(srcr.at[pl.ds(base_row, SEGC)], src_vm)
        pltpu.sync_copy(dstr.at[pl.ds(base_row, SEGC)], dst_vm)
        pltpu.sync_copy(w_hbm.at[pl.ds(base_row, SEGC)], w_vm)

        pltpu.async_copy(h_hbm.at[src_vm.at[0]], bufA, gsemA)

        def pair(p, _):
            j0 = 2 * p
            j1 = j0 + 1
            pltpu.make_async_copy(h_hbm.at[src_vm.at[j0]], bufA, gsemA).wait()
            pltpu.async_copy(h_hbm.at[src_vm.at[j1]], bufB, gsemB)
            _scale_rows(u_vm, 0, w_vm, j0)
            pltpu.sync_copy(bufA, acc_sp.at[dst_vm.at[j0]], add=True)
            pltpu.make_async_copy(h_hbm.at[src_vm.at[j1]], bufB, gsemB).wait()

            @pl.when(p < SEGC // 2 - 1)
            def _prefetch():
                pltpu.async_copy(h_hbm.at[src_vm.at[j0 + 2]], bufA, gsemA)

            _scale_rows(u_vm, 128, w_vm, j1)
            pltpu.sync_copy(bufB, acc_sp.at[dst_vm.at[j1]], add=True)
            return 0

        lax.fori_loop(0, SEGC // 2, pair, 0)

    plsc.subcore_barrier()
    pltpu.sync_copy(
        acc_sp.at[pl.ds(STRIPE * s, STRIPE)],
        acc_hbm.at[pl.ds(STRIPE * s, STRIPE)],
    )


def _sc_body(h00, h01, h10, h11, tabs0, tabd0, tabs1, tabd1, srcr, dstr,
             zeros1d, zeros2d,
             acc00, acc01, acc10, acc11, den0, den1, w0_hbm, w1_hbm,
             src_vm, dst_vm, w_vm, u_vm, den_sp, acc_sp,
             gsemA, gsemB, densem,
             write_den1):
    c = lax.axis_index("c")
    s = lax.axis_index("s")

    @pl.when(c == 0)
    def _core0():
        _phase_weights(s, tabs0, tabd0, w0_hbm, den0, srcr, dstr,
                       src_vm, dst_vm, w_vm, u_vm, zeros1d, den_sp, densem,
                       True)
        _phase_slice(s, h00, acc00, w0_hbm, srcr, dstr,
                     src_vm, dst_vm, w_vm, u_vm, acc_sp, zeros2d, gsemA, gsemB)
        _phase_slice(s, h01, acc01, w0_hbm, srcr, dstr,
                     src_vm, dst_vm, w_vm, u_vm, acc_sp, zeros2d, gsemA, gsemB)

    @pl.when(c == 1)
    def _core1():
        _phase_weights(s, tabs1, tabd1, w1_hbm, den1, srcr, dstr,
                       src_vm, dst_vm, w_vm, u_vm, zeros1d, den_sp, densem,
                       write_den1)
        _phase_slice(s, h10, acc10, w1_hbm, srcr, dstr,
                     src_vm, dst_vm, w_vm, u_vm, acc_sp, zeros2d, gsemA, gsemB)
        _phase_slice(s, h11, acc11, w1_hbm, srcr, dstr,
                     src_vm, dst_vm, w_vm, u_vm, acc_sp, zeros2d, gsemA, gsemB)


def _sc_edge_pass(h_slices, tabs0, tabd0, tabs1, tabd1, srcr, dstr, write_den1):
    mesh = plsc.VectorSubcoreMesh(core_axis_name="c", subcore_axis_name="s")
    f32 = jnp.float32
    zeros1d = jnp.zeros((NP,), f32)
    zeros2d = jnp.zeros((128, 128), f32)
    kern = pl.kernel(
        functools.partial(_sc_body, write_den1=write_den1),
        out_type=[jax.ShapeDtypeStruct((NP, 128), f32)] * 4
        + [jax.ShapeDtypeStruct((NP,), f32)] * 2
        + [jax.ShapeDtypeStruct((EP // 128, 128), f32)] * 2,
        mesh=mesh,
        compiler_params=pltpu.CompilerParams(needs_layout_passes=False),
        scratch_types=[
            pltpu.VMEM((SEGC, 128), jnp.int32),
            pltpu.VMEM((SEGC, 128), jnp.int32),
            pltpu.VMEM((SEGC, 128), f32),
            pltpu.VMEM((256, 128), f32),
            pltpu.VMEM_SHARED((NP,), f32),
            pltpu.VMEM_SHARED((NP, 128), f32),
            pltpu.SemaphoreType.DMA,
            pltpu.SemaphoreType.DMA,
            pltpu.SemaphoreType.DMA,
        ],
    )
    out = kern(h_slices[0], h_slices[1], h_slices[2], h_slices[3],
               tabs0, tabd0, tabs1, tabd1, srcr, dstr, zeros1d, zeros2d)
    return out[:6]


def kernel(x, edge_index, W1, a_src1, a_dst1, b1,
           Wm, a_srcm, a_dstm, bm, Wl, a_srcl, a_dstl, bl):
    src = edge_index[0]
    dst = edge_index[1]
    pad = jnp.zeros((EP - E,), jnp.int32)
    srcr = jnp.concatenate([src, pad]).reshape(EP // 128, 128)
    dstr = jnp.concatenate([dst, pad]).reshape(EP // 128, 128)

    h0, h1, h2, h3, a_s, a_d = _dense1(x, W1, a_src1, a_dst1)

    a_s = a_s.reshape(NROWS, 128)
    a_d = a_d.reshape(NROWS, 128)
    acc0, acc1, acc2, acc3, den1, _ = _sc_edge_pass(
        (h0, h1, h2, h3), a_s, a_d, a_s, a_d, srcr, dstr, False)

    (hm0, hm1, hl0, hl1, ams, amd, als, ald) = _dense2(
        (acc0, acc1, acc2, acc3), den1, b1, Wm, Wl,
        a_srcm, a_dstm, a_srcl, a_dstl)

    ams = ams.reshape(NROWS, 128)
    amd = amd.reshape(NROWS, 128)
    als = als.reshape(NROWS, 128)
    ald = ald.reshape(NROWS, 128)
    am0, am1, al0, al1, den_m, den_l = _sc_edge_pass(
        (hm0, hm1, hl0, hl1), ams, amd, als, ald, srcr, dstr, True)

    mu, logstd = _finalize((am0, am1), (al0, al1), den_m, den_l, bm, bl)
    return (mu, logstd)


# speedup vs baseline: 10.4961x; 1.0217x over previous
import functools

import jax
import jax.numpy as jnp
from jax import lax
from jax.experimental import pallas as pl
from jax.experimental.pallas import tpu as pltpu
from jax.experimental.pallas import tpu_sc as plsc

N = 10000
NP = 10240
NROWS = NP // 128
E = 160000
EP = 163840
CPT = (EP // 128) // 16
STRIPE = NP // 16
DEN_STRIPE = 8

NBLK = 1024
NGRID = NP // NBLK


def _tc1_body(x_ref, w_ref, as_ref, ad_ref, h0, h1, h2, h3, os_ref, od_ref):
    h = jnp.dot(x_ref[...], w_ref[...], preferred_element_type=jnp.float32)
    h0[...] = h[:, 0:128]
    h1[...] = h[:, 128:256]
    h2[...] = h[:, 256:384]
    h3[...] = h[:, 384:512]
    os_ref[...] = jnp.dot(h, as_ref[...], preferred_element_type=jnp.float32)
    od_ref[...] = jnp.dot(h, ad_ref[...], preferred_element_type=jnp.float32)


def _dense1(x, W1, a_src1, a_dst1):
    return pl.pallas_call(
        _tc1_body,
        grid=(NGRID,),
        in_specs=[
            pl.BlockSpec((NBLK, 256), lambda i: (i, 0)),
            pl.BlockSpec((256, 512), lambda i: (0, 0)),
            pl.BlockSpec((512,), lambda i: (0,)),
            pl.BlockSpec((512,), lambda i: (0,)),
        ],
        out_specs=[pl.BlockSpec((NBLK, 128), lambda i: (i, 0))] * 4
        + [pl.BlockSpec((NBLK,), lambda i: (i,))] * 2,
        out_shape=[jax.ShapeDtypeStruct((NP, 128), jnp.float32)] * 4
        + [jax.ShapeDtypeStruct((NP,), jnp.float32)] * 2,
    )(x, W1, a_src1, a_dst1)


def _tc2_body(a0, a1, a2, a3, den_ref, b_ref, wm_ref, wl_ref,
              ams_ref, amd_ref, als_ref, ald_ref,
              hm0, hm1, hl0, hl1, oms, omd, ols, old):
    d = den_ref[...]
    r = jnp.where(d > 0.0, 1.0 / jnp.where(d > 0.0, d, 1.0), 0.0)
    acc = jnp.concatenate([a0[...], a1[...], a2[...], a3[...]], axis=1)
    h = jnp.maximum(acc * r[:, None] + b_ref[...][None, :], 0.0)
    hm = jnp.dot(h, wm_ref[...], preferred_element_type=jnp.float32)
    hl = jnp.dot(h, wl_ref[...], preferred_element_type=jnp.float32)
    hm0[...] = hm[:, 0:128]
    hm1[...] = hm[:, 128:256]
    hl0[...] = hl[:, 0:128]
    hl1[...] = hl[:, 128:256]
    oms[...] = jnp.dot(hm, ams_ref[...], preferred_element_type=jnp.float32)
    omd[...] = jnp.dot(hm, amd_ref[...], preferred_element_type=jnp.float32)
    ols[...] = jnp.dot(hl, als_ref[...], preferred_element_type=jnp.float32)
    old[...] = jnp.dot(hl, ald_ref[...], preferred_element_type=jnp.float32)


def _dense2(acc_slices, den, b1, Wm, Wl, ams, amd, als, ald):
    vec = pl.BlockSpec((256,), lambda i: (0,))
    return pl.pallas_call(
        _tc2_body,
        grid=(NGRID,),
        in_specs=[pl.BlockSpec((NBLK, 128), lambda i: (i, 0))] * 4
        + [
            pl.BlockSpec((NBLK,), lambda i: (i,)),
            pl.BlockSpec((512,), lambda i: (0,)),
            pl.BlockSpec((512, 256), lambda i: (0, 0)),
            pl.BlockSpec((512, 256), lambda i: (0, 0)),
            vec, vec, vec, vec,
        ],
        out_specs=[pl.BlockSpec((NBLK, 128), lambda i: (i, 0))] * 4
        + [pl.BlockSpec((NBLK,), lambda i: (i,))] * 4,
        out_shape=[jax.ShapeDtypeStruct((NP, 128), jnp.float32)] * 4
        + [jax.ShapeDtypeStruct((NP,), jnp.float32)] * 4,
    )(*acc_slices, den, b1, Wm, Wl, ams, amd, als, ald)


def _tc3_body(m0, m1, l0, l1, dm_ref, dl_ref, bm_ref, bl_ref, mu_ref, ls_ref):
    dm = dm_ref[...]
    rm = jnp.where(dm > 0.0, 1.0 / jnp.where(dm > 0.0, dm, 1.0), 0.0)
    dl = dl_ref[...]
    rl = jnp.where(dl > 0.0, 1.0 / jnp.where(dl > 0.0, dl, 1.0), 0.0)
    accm = jnp.concatenate([m0[...], m1[...]], axis=1)
    accl = jnp.concatenate([l0[...], l1[...]], axis=1)
    mu_ref[...] = accm * rm[:, None] + bm_ref[...][None, :]
    ls_ref[...] = accl * rl[:, None] + bl_ref[...][None, :]


def _finalize(am, al, den_m, den_l, bm, bl):
    return pl.pallas_call(
        _tc3_body,
        grid=(NGRID,),
        in_specs=[pl.BlockSpec((NBLK, 128), lambda i: (i, 0))] * 4
        + [pl.BlockSpec((NBLK,), lambda i: (i,))] * 2
        + [pl.BlockSpec((256,), lambda i: (0,))] * 2,
        out_specs=[pl.BlockSpec((NBLK, 256), lambda i: (i, 0))] * 2,
        out_shape=[jax.ShapeDtypeStruct((N, 256), jnp.float32)] * 2,
    )(am[0], am[1], al[0], al[1], den_m, den_l, bm, bl)


SEG = 5
SEGC = CPT // SEG

TABD_OFF = 80


def _phase_weights(s, tabs_hbm, tabd_hbm, w_hbm, den_hbm, srcr, dstr,
                   src_vm, dst_vm, w_vm, u_vm, zeros1d, den_sp, densem,
                   write_den):
    pltpu.sync_copy(tabs_hbm, u_vm.at[pl.ds(0, NROWS)])
    pltpu.sync_copy(tabd_hbm, u_vm.at[pl.ds(TABD_OFF, NROWS)])

    @pl.when(s < 10)
    def _zero_den():
        pltpu.sync_copy(zeros1d.at[pl.ds(1024 * s, 1024)],
                        den_sp.at[pl.ds(1024 * s, 1024)])
    plsc.subcore_barrier()

    for seg in range(SEG):
        base_row = s * CPT + seg * SEGC
        pltpu.sync_copy(srcr.at[pl.ds(base_row, SEGC)], src_vm)
        pltpu.sync_copy(dstr.at[pl.ds(base_row, SEGC)], dst_vm)

        def chunk(j, _):
            base_e = (base_row + j) * 128

            def vec(k, _):
                si = src_vm[j, pl.ds(k * 16, 16)]
                di = dst_vm[j, pl.ds(k * 16, 16)]
                av = plsc.load_gather(
                    u_vm, [jnp.right_shift(si, 7), jnp.bitwise_and(si, 127)])
                bv = plsc.load_gather(
                    u_vm, [jnp.right_shift(di, 7) + TABD_OFF,
                           jnp.bitwise_and(di, 127)])
                e = av + bv
                e = jnp.where(e > 0.0, e, 0.2 * e)
                w = jnp.exp(e)
                eid = base_e + k * 16 + lax.iota(jnp.int32, 16)
                w = jnp.where(eid < E, w, 0.0)
                w_vm[j, pl.ds(k * 16, 16)] = w
                return 0

            lax.fori_loop(0, 8, vec, 0)
            pltpu.async_copy(w_vm.at[j], den_sp.at[dst_vm.at[j]], densem,
                             add=True)
            return 0

        lax.fori_loop(0, SEGC, chunk, 0)

        def drain(j, _):
            pltpu.make_async_copy(w_vm.at[j], den_sp.at[dst_vm.at[j]],
                                  densem).wait()
            return 0

        lax.fori_loop(0, SEGC, drain, 0)
        pltpu.sync_copy(w_vm, w_hbm.at[pl.ds(base_row, SEGC)])

    plsc.subcore_barrier()
    if write_den:
        @pl.when(s < 10)
        def _write_den():
            pltpu.sync_copy(den_sp.at[pl.ds(1024 * s, 1024)],
                            den_hbm.at[pl.ds(1024 * s, 1024)])


def _scale_rows(u_vm, boff, w_vm, j):
    def sgroup(g, _):
        wv = w_vm[j, pl.ds(g * 16, 16)]
        for i in range(16):
            row = g * 16 + i
            wvi = jnp.full((16,), wv[i], jnp.float32)
            for k in range(8):
                sl = pl.ds(k * 16, 16)
                u_vm[boff + row, sl] = u_vm[boff + row, sl] * wvi
        return 0

    lax.fori_loop(0, 8, sgroup, 0)


def _phase_slice(s, h_hbm, acc_hbm, w_hbm, srcr, dstr,
                 src_vm, dst_vm, w_vm, u_vm, acc_sp, zeros2d,
                 gsemA, gsemB, ssemA, ssemB):
    bufA = u_vm.at[pl.ds(0, 128)]
    bufB = u_vm.at[pl.ds(128, 128)]
    for z in range(STRIPE // 128):
        pltpu.sync_copy(zeros2d, acc_sp.at[pl.ds(STRIPE * s + 128 * z, 128)])
    plsc.subcore_barrier()

    for seg in range(SEG):
        base_row = s * CPT + seg * SEGC
        pltpu.sync_copy(srcr.at[pl.ds(base_row, SEGC)], src_vm)
        pltpu.sync_copy(dstr.at[pl.ds(base_row, SEGC)], dst_vm)
        pltpu.sync_copy(w_hbm.at[pl.ds(base_row, SEGC)], w_vm)

        pltpu.async_copy(h_hbm.at[src_vm.at[0]], bufA, gsemA)

        def pair(p, _):
            j0 = 2 * p
            j1 = j0 + 1

            pltpu.async_copy(h_hbm.at[src_vm.at[j1]], bufB, gsemB)
            pltpu.make_async_copy(h_hbm.at[src_vm.at[j0]], bufA, gsemA).wait()
            _scale_rows(u_vm, 0, w_vm, j0)
            pltpu.async_copy(bufA, acc_sp.at[dst_vm.at[j0]], ssemA, add=True)
            pltpu.make_async_copy(bufA, acc_sp.at[dst_vm.at[j0]], ssemA).wait()

            @pl.when(p < SEGC // 2 - 1)
            def _prefetch():
                pltpu.async_copy(h_hbm.at[src_vm.at[j0 + 2]], bufA, gsemA)

            pltpu.make_async_copy(h_hbm.at[src_vm.at[j1]], bufB, gsemB).wait()
            _scale_rows(u_vm, 128, w_vm, j1)
            pltpu.async_copy(bufB, acc_sp.at[dst_vm.at[j1]], ssemB, add=True)
            pltpu.make_async_copy(bufB, acc_sp.at[dst_vm.at[j1]], ssemB).wait()
            return 0

        lax.fori_loop(0, SEGC // 2, pair, 0)

    plsc.subcore_barrier()
    pltpu.sync_copy(
        acc_sp.at[pl.ds(STRIPE * s, STRIPE)],
        acc_hbm.at[pl.ds(STRIPE * s, STRIPE)],
    )


def _sc_body(h00, h01, h10, h11, tabs0, tabd0, tabs1, tabd1, srcr, dstr,
             zeros1d, zeros2d,
             acc00, acc01, acc10, acc11, den0, den1, w0_hbm, w1_hbm,
             src_vm, dst_vm, w_vm, u_vm, den_sp, acc_sp,
             gsemA, gsemB, ssemA, ssemB, densem,
             write_den1):
    c = lax.axis_index("c")
    s = lax.axis_index("s")

    @pl.when(c == 0)
    def _core0():
        _phase_weights(s, tabs0, tabd0, w0_hbm, den0, srcr, dstr,
                       src_vm, dst_vm, w_vm, u_vm, zeros1d, den_sp, densem,
                       True)
        _phase_slice(s, h00, acc00, w0_hbm, srcr, dstr,
                     src_vm, dst_vm, w_vm, u_vm, acc_sp, zeros2d,
                     gsemA, gsemB, ssemA, ssemB)
        _phase_slice(s, h01, acc01, w0_hbm, srcr, dstr,
                     src_vm, dst_vm, w_vm, u_vm, acc_sp, zeros2d,
                     gsemA, gsemB, ssemA, ssemB)

    @pl.when(c == 1)
    def _core1():
        _phase_weights(s, tabs1, tabd1, w1_hbm, den1, srcr, dstr,
                       src_vm, dst_vm, w_vm, u_vm, zeros1d, den_sp, densem,
                       write_den1)
        _phase_slice(s, h10, acc10, w1_hbm, srcr, dstr,
                     src_vm, dst_vm, w_vm, u_vm, acc_sp, zeros2d,
                     gsemA, gsemB, ssemA, ssemB)
        _phase_slice(s, h11, acc11, w1_hbm, srcr, dstr,
                     src_vm, dst_vm, w_vm, u_vm, acc_sp, zeros2d,
                     gsemA, gsemB, ssemA, ssemB)


def _sc_edge_pass(h_slices, tabs0, tabd0, tabs1, tabd1, srcr, dstr, write_den1):
    mesh = plsc.VectorSubcoreMesh(core_axis_name="c", subcore_axis_name="s")
    f32 = jnp.float32
    zeros1d = jnp.zeros((NP,), f32)
    zeros2d = jnp.zeros((128, 128), f32)
    kern = pl.kernel(
        functools.partial(_sc_body, write_den1=write_den1),
        out_type=[jax.ShapeDtypeStruct((NP, 128), f32)] * 4
        + [jax.ShapeDtypeStruct((NP,), f32)] * 2
        + [jax.ShapeDtypeStruct((EP // 128, 128), f32)] * 2,
        mesh=mesh,
        compiler_params=pltpu.CompilerParams(needs_layout_passes=False),
        scratch_types=[
            pltpu.VMEM((SEGC, 128), jnp.int32),
            pltpu.VMEM((SEGC, 128), jnp.int32),
            pltpu.VMEM((SEGC, 128), f32),
            pltpu.VMEM((256, 128), f32),
            pltpu.VMEM_SHARED((NP,), f32),
            pltpu.VMEM_SHARED((NP, 128), f32),
            pltpu.SemaphoreType.DMA,
            pltpu.SemaphoreType.DMA,
            pltpu.SemaphoreType.DMA,
            pltpu.SemaphoreType.DMA,
            pltpu.SemaphoreType.DMA,
        ],
    )
    out = kern(h_slices[0], h_slices[1], h_slices[2], h_slices[3],
               tabs0, tabd0, tabs1, tabd1, srcr, dstr, zeros1d, zeros2d)
    return out[:6]


def kernel(x, edge_index, W1, a_src1, a_dst1, b1,
           Wm, a_srcm, a_dstm, bm, Wl, a_srcl, a_dstl, bl):
    src = edge_index[0]
    dst = edge_index[1]
    pad = jnp.zeros((EP - E,), jnp.int32)
    srcr = jnp.concatenate([src, pad]).reshape(EP // 128, 128)
    dstr = jnp.concatenate([dst, pad]).reshape(EP // 128, 128)

    h0, h1, h2, h3, a_s, a_d = _dense1(x, W1, a_src1, a_dst1)

    a_s = a_s.reshape(NROWS, 128)
    a_d = a_d.reshape(NROWS, 128)
    acc0, acc1, acc2, acc3, den1, _ = _sc_edge_pass(
        (h0, h1, h2, h3), a_s, a_d, a_s, a_d, srcr, dstr, False)

    (hm0, hm1, hl0, hl1, ams, amd, als, ald) = _dense2(
        (acc0, acc1, acc2, acc3), den1, b1, Wm, Wl,
        a_srcm, a_dstm, a_srcl, a_dstl)

    ams = ams.reshape(NROWS, 128)
    amd = amd.reshape(NROWS, 128)
    als = als.reshape(NROWS, 128)
    ald = ald.reshape(NROWS, 128)
    am0, am1, al0, al1, den_m, den_l = _sc_edge_pass(
        (hm0, hm1, hl0, hl1), ams, amd, als, ald, srcr, dstr, True)

    mu, logstd = _finalize((am0, am1), (al0, al1), den_m, den_l, bm, bl)
    return (mu, logstd)
